# baseline (speedup 1.0000x reference)
import math

import jax
import jax.numpy as jnp
from jax import lax
from jax.experimental import pallas as pl
from jax.experimental.pallas import tpu as pltpu
from jax.experimental.pallas import tpu_sc as plsc

VOCAB = 1000000
D = 64
NT = 4096
NS_ = 200
B = NT * NS_
SCALE = math.sqrt(D)

_info = plsc.get_sparse_core_info()
NC, NSUB, L = _info.num_cores, _info.num_subcores, _info.num_lanes
NW = NC * NSUB

FULL_BLOCKS = VOCAB // 128
BPW_BASE = FULL_BLOCKS // NW
BPW_EXTRA = FULL_BLOCKS - BPW_BASE * NW


def _repack_body(wt_hbm, wtail_hbm, tab_hbm, blk_v, tb_v, semg, sems):
    wid = lax.axis_index("s") * NC + lax.axis_index("c")
    iota = jax.lax.iota(jnp.int32, L)

    def fire_load(i, b):
        bl = wid + i * NW
        pltpu.async_copy(
            wt_hbm.at[:, pl.ds(pl.multiple_of(bl * 128, 128), 128)],
            blk_v.at[b], semg.at[b],
        )

    def transpose_blk(b):
        @plsc.parallel_loop(0, 64, unroll=8)
        def transpose_pair(r2):
            for rr in range(2):
                rv = jnp.full((L,), r2 * 2 + rr, jnp.int32)
                for j0 in range(D // L):
                    v = plsc.load_gather(blk_v.at[b], [iota + j0 * L, rv])
                    tb_v[b, r2, pl.ds(rr * D + j0 * L, L)] = v

    n_mine = jnp.where(wid < BPW_EXTRA, BPW_BASE + 1, BPW_BASE).astype(jnp.int32)

    fire_load(0, 0)

    def block_step(i, _):
        b = i & 1
        bl = wid + i * NW

        @pl.when(i + 1 < n_mine)
        def _():
            fire_load(i + 1, 1 - b)

        pltpu.make_async_copy(
            wt_hbm.at[:, pl.ds(0, 128)], blk_v.at[b], semg.at[b]
        ).wait()

        @pl.when(i >= 2)
        def _():
            pltpu.make_async_copy(
                tab_hbm.at[pl.ds(0, 64)], tb_v.at[b], sems.at[b]
            ).wait()

        transpose_blk(b)
        pltpu.async_copy(
            tb_v.at[b], tab_hbm.at[pl.ds(pl.multiple_of(bl * 64, 64), 64)],
            sems.at[b],
        )
        return ()

    lax.fori_loop(0, n_mine, block_step, ())

    for b in range(2):
        pltpu.make_async_copy(
            tab_hbm.at[pl.ds(0, 64)], tb_v.at[b], sems.at[b]
        ).wait()

    @pl.when(wid == NW - 1)
    def _():
        pltpu.sync_copy(wtail_hbm, blk_v.at[0])
        transpose_blk(0)
        pltpu.sync_copy(tb_v.at[0, pl.ds(0, 32)], tab_hbm.at[pl.ds(VOCAB // 2 - 32, 32)])


N_CHUNKS = B // 128
CPW = N_CHUNKS // NW


def _lookup_body(xt_hbm, tab_hbm, out_hbm, idx_v, g_v, tb_v, semi, semg, sems):
    wid = lax.axis_index("s") * NC + lax.axis_index("c")
    iota = jax.lax.iota(jnp.int32, L)
    c0 = wid * CPW

    def fire_idx(i, b):
        c = c0 + i
        pltpu.async_copy(
            xt_hbm.at[c // 32, pl.ds(pl.multiple_of((c % 32) * 128, 128), 128)],
            idx_v.at[b], semi.at[b],
        )

    fire_idx(0, 0)

    def step(i, _):
        b = i & 1
        p = (i - 1) & 1

        @pl.when(i < CPW)
        def _():
            pltpu.make_async_copy(
                xt_hbm.at[0, pl.ds(0, 128)], idx_v.at[b], semi.at[b]
            ).wait()
            pltpu.async_copy(tab_hbm.at[idx_v.at[b]], g_v.at[b], semg.at[b])

        @pl.when(i >= 1)
        def _():
            pltpu.make_async_copy(
                tab_hbm.at[pl.ds(0, 128)], g_v.at[p], semg.at[p]
            ).wait()

        @pl.when(i + 1 < CPW)
        def _():
            fire_idx(i + 1, p)

        @pl.when(i >= 1)
        def _():
            c = c0 + i - 1

            @pl.when(i - 1 >= 2)
            def _():
                pltpu.make_async_copy(
                    out_hbm.at[0, :, 0], tb_v.at[p], sems.at[p]
                ).wait()

            for bl0 in range(8):
                rows = iota + bl0 * L

                @plsc.parallel_loop(0, 8, unroll=4)
                def emit_jg(jg):
                    for jr in range(8):
                        jv = jnp.full((L,), jg * 8 + jr, jnp.int32)
                        v = plsc.load_gather(g_v.at[p], [rows, jv]) * SCALE
                        tb_v[p, jg, jr, pl.ds(bl0 * L, L)] = v
            pltpu.async_copy(tb_v.at[p], out_hbm.at[c // 32, :, c % 32], sems.at[p])

        return ()

    lax.fori_loop(0, CPW + 1, step, ())

    for b in range(2):
        pltpu.make_async_copy(
            out_hbm.at[0, :, 0], tb_v.at[b], sems.at[b]
        ).wait()


@jax.jit
def _embed(xt, wt, wtail):
    mesh = plsc.VectorSubcoreMesh(core_axis_name="c", subcore_axis_name="s")
    repack = pl.kernel(
        _repack_body,
        out_type=jax.ShapeDtypeStruct((VOCAB // 2, 128), jnp.float32),
        mesh=mesh,
        scratch_types=[
            pltpu.VMEM((2, D, 128), jnp.float32),
            pltpu.VMEM((2, D, 128), jnp.float32),
            pltpu.SemaphoreType.DMA((2,)),
            pltpu.SemaphoreType.DMA((2,)),
        ],
        compiler_params=pltpu.CompilerParams(use_tc_tiling_on_sc=True, needs_layout_passes=False),
    )
    tab = repack(wt, wtail)
    tabl = tab.reshape(VOCAB, D)
    lookup = pl.kernel(
        _lookup_body,
        out_type=jax.ShapeDtypeStruct((NS_, 8, 32, 8, 128), jnp.float32),
        mesh=mesh,
        scratch_types=[
            pltpu.VMEM((2, 128), jnp.int32),
            pltpu.VMEM((2, 128, D), jnp.float32),
            pltpu.VMEM((2, 8, 8, 128), jnp.float32),
            pltpu.SemaphoreType.DMA((2,)),
            pltpu.SemaphoreType.DMA((2,)),
            pltpu.SemaphoreType.DMA((2,)),
        ],
        compiler_params=pltpu.CompilerParams(use_tc_tiling_on_sc=False, needs_layout_passes=False),
    )
    return lookup(xt, tabl)


def kernel(x, embed_weight):
    xt = x.astype(jnp.int32).T
    wt = embed_weight.T
    wtail = jnp.pad(embed_weight[VOCAB - 64:].T, ((0, 0), (0, 64)))
    out5 = _embed(xt, wt, wtail)
    return out5.transpose(2, 4, 0, 1, 3).reshape(NT, NS_, D)


# speedup vs baseline: 3.5337x; 3.5337x over previous
import math

import jax
import jax.numpy as jnp
from jax import lax
from jax.experimental import pallas as pl
from jax.experimental.pallas import tpu as pltpu
from jax.experimental.pallas import tpu_sc as plsc

VOCAB = 1000000
D = 64
NT = 4096
NS_ = 200
B = NT * NS_
SCALE = math.sqrt(D)

_info = plsc.get_sparse_core_info()
NC, NSUB, L = _info.num_cores, _info.num_subcores, _info.num_lanes
NW = NC * NSUB

FULL_BLOCKS = VOCAB // 128
BPW_BASE = FULL_BLOCKS // NW
BPW_EXTRA = FULL_BLOCKS - BPW_BASE * NW


def _repack_body(wt_hbm, wtail_hbm, tab_hbm, blk_v, tb_v, semg, sems):
    wid = lax.axis_index("s") * NC + lax.axis_index("c")
    iota = jax.lax.iota(jnp.int32, L)

    def fire_load(i, b):
        bl = wid + i * NW
        pltpu.async_copy(
            wt_hbm.at[:, pl.ds(pl.multiple_of(bl * 128, 128), 128)],
            blk_v.at[b], semg.at[b],
        )

    def transpose_blk(b):
        for j0 in range(D // L):
            jlanes = iota + j0 * L

            @plsc.parallel_loop(0, 128, unroll=4)
            def transpose_col(r):
                cv = (iota + r) & 127
                v = plsc.load_gather(blk_v.at[b], [jlanes, cv])
                plsc.store_scatter(
                    tb_v.at[b],
                    [jax.lax.shift_right_logical(cv, 1), (cv & 1) * D + jlanes],
                    v,
                )

    n_mine = jnp.where(wid < BPW_EXTRA, BPW_BASE + 1, BPW_BASE).astype(jnp.int32)

    fire_load(0, 0)

    def block_step(i, _):
        b = i & 1
        bl = wid + i * NW

        @pl.when(i + 1 < n_mine)
        def _():
            fire_load(i + 1, 1 - b)

        pltpu.make_async_copy(
            wt_hbm.at[:, pl.ds(0, 128)], blk_v.at[b], semg.at[b]
        ).wait()

        @pl.when(i >= 2)
        def _():
            pltpu.make_async_copy(
                tab_hbm.at[pl.ds(0, 64)], tb_v.at[b], sems.at[b]
            ).wait()

        transpose_blk(b)
        pltpu.async_copy(
            tb_v.at[b], tab_hbm.at[pl.ds(pl.multiple_of(bl * 64, 64), 64)],
            sems.at[b],
        )
        return ()

    lax.fori_loop(0, n_mine, block_step, ())

    for b in range(2):
        pltpu.make_async_copy(
            tab_hbm.at[pl.ds(0, 64)], tb_v.at[b], sems.at[b]
        ).wait()

    @pl.when(wid == NW - 1)
    def _():
        pltpu.sync_copy(wtail_hbm, blk_v.at[0])
        transpose_blk(0)
        pltpu.sync_copy(tb_v.at[0, pl.ds(0, 32)], tab_hbm.at[pl.ds(VOCAB // 2 - 32, 32)])


N_CHUNKS = B // 128
CPW = N_CHUNKS // NW


def _lookup_body(xt_hbm, tab_hbm, out_hbm, idx_v, g_v, tb_v, semi, semg, sems):
    wid = lax.axis_index("s") * NC + lax.axis_index("c")
    iota = jax.lax.iota(jnp.int32, L)
    c0 = wid * CPW

    def fire_idx(i, b):
        c = c0 + i
        pltpu.async_copy(
            xt_hbm.at[c // 32, pl.ds(pl.multiple_of((c % 32) * 128, 128), 128)],
            idx_v.at[b], semi.at[b],
        )

    fire_idx(0, 0)

    def step(i, _):
        b = i & 1
        p = (i - 1) & 1

        @pl.when(i < CPW)
        def _():
            pltpu.make_async_copy(
                xt_hbm.at[0, pl.ds(0, 128)], idx_v.at[b], semi.at[b]
            ).wait()
            pltpu.async_copy(tab_hbm.at[idx_v.at[b]], g_v.at[b], semg.at[b])

        @pl.when(i >= 1)
        def _():
            pltpu.make_async_copy(
                tab_hbm.at[pl.ds(0, 128)], g_v.at[p], semg.at[p]
            ).wait()

        @pl.when(i + 1 < CPW)
        def _():
            fire_idx(i + 1, p)

        @pl.when(i >= 1)
        def _():
            c = c0 + i - 1

            @pl.when(i - 1 >= 2)
            def _():
                pltpu.make_async_copy(
                    out_hbm.at[0, :, 0], tb_v.at[p], sems.at[p]
                ).wait()

            for bl0 in range(8):
                rows = iota + bl0 * L

                @plsc.parallel_loop(0, D, unroll=4)
                def emit_j(j):
                    jv = (iota + j) & (D - 1)
                    v = plsc.load_gather(g_v.at[p], [rows, jv]) * SCALE
                    plsc.store_scatter(
                        tb_v.at[p],
                        [jax.lax.shift_right_logical(jv, 3), jv & 7, rows],
                        v,
                    )
            pltpu.async_copy(tb_v.at[p], out_hbm.at[c // 32, :, c % 32], sems.at[p])

        return ()

    lax.fori_loop(0, CPW + 1, step, ())

    for b in range(2):
        pltpu.make_async_copy(
            out_hbm.at[0, :, 0], tb_v.at[b], sems.at[b]
        ).wait()


@jax.jit
def _embed(xt, wt, wtail):
    mesh = plsc.VectorSubcoreMesh(core_axis_name="c", subcore_axis_name="s")
    repack = pl.kernel(
        _repack_body,
        out_type=jax.ShapeDtypeStruct((VOCAB // 2, 128), jnp.float32),
        mesh=mesh,
        scratch_types=[
            pltpu.VMEM((2, D, 128), jnp.float32),
            pltpu.VMEM((2, D, 128), jnp.float32),
            pltpu.SemaphoreType.DMA((2,)),
            pltpu.SemaphoreType.DMA((2,)),
        ],
        compiler_params=pltpu.CompilerParams(use_tc_tiling_on_sc=True, needs_layout_passes=False),
    )
    tab = repack(wt, wtail)
    tabl = tab.reshape(VOCAB, D)
    lookup = pl.kernel(
        _lookup_body,
        out_type=jax.ShapeDtypeStruct((NS_, 8, 32, 8, 128), jnp.float32),
        mesh=mesh,
        scratch_types=[
            pltpu.VMEM((2, 128), jnp.int32),
            pltpu.VMEM((2, 128, D), jnp.float32),
            pltpu.VMEM((2, 8, 8, 128), jnp.float32),
            pltpu.SemaphoreType.DMA((2,)),
            pltpu.SemaphoreType.DMA((2,)),
            pltpu.SemaphoreType.DMA((2,)),
        ],
        compiler_params=pltpu.CompilerParams(use_tc_tiling_on_sc=False, needs_layout_passes=False),
    )
    return lookup(xt, tabl)


def kernel(x, embed_weight):
    xt = x.astype(jnp.int32).T
    wt = embed_weight.T
    wtail = jnp.pad(embed_weight[VOCAB - 64:].T, ((0, 0), (0, 64)))
    out5 = _embed(xt, wt, wtail)
    return out5.transpose(2, 4, 0, 1, 3).reshape(NT, NS_, D)


# speedup vs baseline: 3.9003x; 1.1037x over previous
import math

import jax
import jax.numpy as jnp
from jax import lax
from jax.experimental import pallas as pl
from jax.experimental.pallas import tpu as pltpu
from jax.experimental.pallas import tpu_sc as plsc

VOCAB = 1000000
D = 64
NT = 4096
NS_ = 200
B = NT * NS_
SCALE = math.sqrt(D)

_info = plsc.get_sparse_core_info()
NC, NSUB, L = _info.num_cores, _info.num_subcores, _info.num_lanes
NW = NC * NSUB

FULL_BLOCKS = VOCAB // 128
BPW_BASE = FULL_BLOCKS // NW
BPW_EXTRA = FULL_BLOCKS - BPW_BASE * NW


def _repack_body(wt_hbm, wtail_hbm, tab_hbm, blk_v, tb_v, semg, sems):
    wid = lax.axis_index("s") * NC + lax.axis_index("c")
    iota = jax.lax.iota(jnp.int32, L)

    def fire_load(i, b):
        bl = wid + i * NW
        pltpu.async_copy(
            wt_hbm.at[:, pl.ds(pl.multiple_of(bl * 128, 128), 128)],
            blk_v.at[b], semg.at[b],
        )

    def transpose_blk(b):
        for j0 in range(D // L):
            jlanes = iota + j0 * L

            @plsc.parallel_loop(0, 128, unroll=4)
            def transpose_col(r):
                cv = (iota + r) & 127
                v = plsc.load_gather(blk_v.at[b], [jlanes, cv])
                plsc.store_scatter(
                    tb_v.at[b],
                    [jax.lax.shift_right_logical(cv, 1), (cv & 1) * D + jlanes],
                    v,
                )

    n_mine = jnp.where(wid < BPW_EXTRA, BPW_BASE + 1, BPW_BASE).astype(jnp.int32)

    fire_load(0, 0)

    def block_step(i, _):
        b = i & 1
        bl = wid + i * NW

        @pl.when(i + 1 < n_mine)
        def _():
            fire_load(i + 1, 1 - b)

        pltpu.make_async_copy(
            wt_hbm.at[:, pl.ds(0, 128)], blk_v.at[b], semg.at[b]
        ).wait()

        @pl.when(i >= 2)
        def _():
            pltpu.make_async_copy(
                tab_hbm.at[pl.ds(0, 64)], tb_v.at[b], sems.at[b]
            ).wait()

        transpose_blk(b)
        pltpu.async_copy(
            tb_v.at[b], tab_hbm.at[pl.ds(pl.multiple_of(bl * 64, 64), 64)],
            sems.at[b],
        )
        return ()

    lax.fori_loop(0, n_mine, block_step, ())

    for b in range(2):
        pltpu.make_async_copy(
            tab_hbm.at[pl.ds(0, 64)], tb_v.at[b], sems.at[b]
        ).wait()

    @pl.when(wid == NW - 1)
    def _():
        pltpu.sync_copy(wtail_hbm, blk_v.at[0])
        transpose_blk(0)
        pltpu.sync_copy(tb_v.at[0, pl.ds(0, 32)], tab_hbm.at[pl.ds(VOCAB // 2 - 32, 32)])


N_CHUNKS = B // 128
CPW = N_CHUNKS // NW


def _lookup_body(xt_hbm, tab_hbm, out_hbm, idx_v, g_v, tb_v, semg, sems):
    wid = lax.axis_index("s") * NC + lax.axis_index("c")
    iota = jax.lax.iota(jnp.int32, L)
    c0 = wid * CPW

    pltpu.sync_copy(xt_hbm.at[pl.ds(pl.multiple_of(c0 * 128, 128), CPW * 128)], idx_v)

    def fire_gather(i, g):
        pltpu.async_copy(
            tab_hbm.at[idx_v.at[pl.ds(pl.multiple_of(i * 128, 128), 128)]],
            g_v.at[g], semg.at[g],
        )

    fire_gather(0, 0)
    fire_gather(1, 1)

    def step(i, _):
        g = i % 3
        b = i & 1

        @pl.when(i + 2 < CPW)
        def _():
            fire_gather(i + 2, (i + 2) % 3)

        pltpu.make_async_copy(
            tab_hbm.at[pl.ds(0, 128)], g_v.at[g], semg.at[g]
        ).wait()

        @pl.when(i >= 2)
        def _():
            pltpu.make_async_copy(
                out_hbm.at[0, :, 0], tb_v.at[b], sems.at[b]
            ).wait()

        for bl0 in range(8):
            rows = iota + bl0 * L

            @plsc.parallel_loop(0, D, unroll=4)
            def emit_j(j):
                jv = (iota + j) & (D - 1)
                v = plsc.load_gather(g_v.at[g], [rows, jv]) * SCALE
                plsc.store_scatter(
                    tb_v.at[b],
                    [jax.lax.shift_right_logical(jv, 3), jv & 7, rows],
                    v,
                )
        c = c0 + i
        pltpu.async_copy(tb_v.at[b], out_hbm.at[c // 32, :, c % 32], sems.at[b])
        return ()

    lax.fori_loop(0, CPW, step, ())

    for b in range(2):
        pltpu.make_async_copy(
            out_hbm.at[0, :, 0], tb_v.at[b], sems.at[b]
        ).wait()


@jax.jit
def _embed(xt, wt, wtail):
    mesh = plsc.VectorSubcoreMesh(core_axis_name="c", subcore_axis_name="s")
    repack = pl.kernel(
        _repack_body,
        out_type=jax.ShapeDtypeStruct((VOCAB // 2, 128), jnp.float32),
        mesh=mesh,
        scratch_types=[
            pltpu.VMEM((2, D, 128), jnp.float32),
            pltpu.VMEM((2, D, 128), jnp.float32),
            pltpu.SemaphoreType.DMA((2,)),
            pltpu.SemaphoreType.DMA((2,)),
        ],
        compiler_params=pltpu.CompilerParams(use_tc_tiling_on_sc=True, needs_layout_passes=False),
    )
    tab = repack(wt, wtail)
    tabl = tab.reshape(VOCAB, D)
    lookup = pl.kernel(
        _lookup_body,
        out_type=jax.ShapeDtypeStruct((NS_, 8, 32, 8, 128), jnp.float32),
        mesh=mesh,
        scratch_types=[
            pltpu.VMEM((CPW * 128,), jnp.int32),
            pltpu.VMEM((3, 128, D), jnp.float32),
            pltpu.VMEM((2, 8, 8, 128), jnp.float32),
            pltpu.SemaphoreType.DMA((3,)),
            pltpu.SemaphoreType.DMA((2,)),
        ],
        compiler_params=pltpu.CompilerParams(use_tc_tiling_on_sc=False, needs_layout_passes=False),
    )
    return lookup(xt, tabl)


def kernel(x, embed_weight):
    xt = x.astype(jnp.int32).T.reshape(B)
    wt = embed_weight.T
    wtail = jnp.pad(embed_weight[VOCAB - 64:].T, ((0, 0), (0, 64)))
    out5 = _embed(xt, wt, wtail)
    return out5.transpose(2, 4, 0, 1, 3).reshape(NT, NS_, D)


# speedup vs baseline: 4.4327x; 1.1365x over previous
import math

import jax
import jax.numpy as jnp
from jax import lax
from jax.experimental import pallas as pl
from jax.experimental.pallas import tpu as pltpu
from jax.experimental.pallas import tpu_sc as plsc

VOCAB = 1000000
D = 64
NT = 4096
NS_ = 200
B = NT * NS_
SCALE = math.sqrt(D)

_info = plsc.get_sparse_core_info()
NC, NSUB, L = _info.num_cores, _info.num_subcores, _info.num_lanes
NW = NC * NSUB

FULL_BLOCKS = VOCAB // 128
BPW_BASE = FULL_BLOCKS // NW
BPW_EXTRA = FULL_BLOCKS - BPW_BASE * NW


def _repack_body(wt_hbm, wtail_hbm, tab_hbm, blk_v, tb_v, semg, sems):
    wid = lax.axis_index("s") * NC + lax.axis_index("c")
    iota = jax.lax.iota(jnp.int32, L)

    def fire_load(i, b):
        bl = wid + i * NW
        pltpu.async_copy(
            wt_hbm.at[:, pl.ds(pl.multiple_of(bl * 128, 128), 128)],
            blk_v.at[b], semg.at[b],
        )

    def transpose_blk(b, tb):
        for j0 in range(D // L):
            jlanes = iota + j0 * L

            @plsc.parallel_loop(0, 128, unroll=4)
            def transpose_col(r):
                cv = (iota + r) & 127
                v = plsc.load_gather(blk_v.at[b], [jlanes, cv])
                plsc.store_scatter(
                    tb_v.at[tb],
                    [jax.lax.shift_right_logical(cv, 1), (cv & 1) * D + jlanes],
                    v,
                )

    n_mine = jnp.where(wid < BPW_EXTRA, BPW_BASE + 1, BPW_BASE).astype(jnp.int32)

    fire_load(0, 0)

    @pl.when(1 < n_mine)
    def _():
        fire_load(1, 1)

    def block_step(i, _):
        b = i % 3
        bl = wid + i * NW

        @pl.when(i + 2 < n_mine)
        def _():
            fire_load(i + 2, (i + 2) % 3)

        pltpu.make_async_copy(
            wt_hbm.at[:, pl.ds(0, 128)], blk_v.at[b], semg.at[b]
        ).wait()

        tb = i & 1

        @pl.when(i >= 2)
        def _():
            pltpu.make_async_copy(
                tab_hbm.at[pl.ds(0, 64)], tb_v.at[tb], sems.at[tb]
            ).wait()

        transpose_blk(b, tb)
        pltpu.async_copy(
            tb_v.at[tb], tab_hbm.at[pl.ds(pl.multiple_of(bl * 64, 64), 64)],
            sems.at[tb],
        )
        return ()

    lax.fori_loop(0, n_mine, block_step, ())

    for b in range(2):
        pltpu.make_async_copy(
            tab_hbm.at[pl.ds(0, 64)], tb_v.at[b], sems.at[b]
        ).wait()

    @pl.when(wid == NW - 1)
    def _():
        pltpu.sync_copy(wtail_hbm, blk_v.at[0])
        transpose_blk(0, 0)
        pltpu.sync_copy(tb_v.at[0, pl.ds(0, 32)], tab_hbm.at[pl.ds(VOCAB // 2 - 32, 32)])


N_CHUNKS = B // 128
CPW = N_CHUNKS // NW


def _lookup_body(xt_hbm, tab_hbm, out_hbm, idx_v, g_v, tb_v, semg, sems):
    wid = lax.axis_index("s") * NC + lax.axis_index("c")
    iota = jax.lax.iota(jnp.int32, L)
    c0 = wid * CPW

    pltpu.sync_copy(xt_hbm.at[pl.ds(pl.multiple_of(c0 * 128, 128), CPW * 128)], idx_v)

    def fire_gather(i, g):
        pltpu.async_copy(
            tab_hbm.at[idx_v.at[pl.ds(pl.multiple_of(i * 128, 128), 128)]],
            g_v.at[g], semg.at[g],
        )

    fire_gather(0, 0)
    fire_gather(1, 1)

    def step(i, _):
        g = i % 3
        b = i & 1

        @pl.when(i + 2 < CPW)
        def _():
            fire_gather(i + 2, (i + 2) % 3)

        pltpu.make_async_copy(
            tab_hbm.at[pl.ds(0, 128)], g_v.at[g], semg.at[g]
        ).wait()

        @pl.when(i >= 2)
        def _():
            pltpu.make_async_copy(
                out_hbm.at[0, :, 0], tb_v.at[b], sems.at[b]
            ).wait()

        for bl0 in range(8):
            rows = iota + bl0 * L

            @plsc.parallel_loop(0, D, unroll=4)
            def emit_j(j):
                jv = (iota + j) & (D - 1)
                v = plsc.load_gather(g_v.at[g], [rows, jv]) * SCALE
                plsc.store_scatter(
                    tb_v.at[b],
                    [jax.lax.shift_right_logical(jv, 3), jv & 7, rows],
                    v,
                )
        c = c0 + i
        pltpu.async_copy(tb_v.at[b], out_hbm.at[c // 32, :, c % 32], sems.at[b])
        return ()

    lax.fori_loop(0, CPW, step, ())

    for b in range(2):
        pltpu.make_async_copy(
            out_hbm.at[0, :, 0], tb_v.at[b], sems.at[b]
        ).wait()


@jax.jit
def _embed(xt, wt, wtail):
    mesh = plsc.VectorSubcoreMesh(core_axis_name="c", subcore_axis_name="s")
    repack = pl.kernel(
        _repack_body,
        out_type=jax.ShapeDtypeStruct((VOCAB // 2, 128), jnp.float32),
        mesh=mesh,
        scratch_types=[
            pltpu.VMEM((3, D, 128), jnp.float32),
            pltpu.VMEM((2, D, 128), jnp.float32),
            pltpu.SemaphoreType.DMA((3,)),
            pltpu.SemaphoreType.DMA((2,)),
        ],
        compiler_params=pltpu.CompilerParams(use_tc_tiling_on_sc=True, needs_layout_passes=False),
    )
    tab = repack(wt, wtail)
    tabl = tab.reshape(VOCAB, D)
    lookup = pl.kernel(
        _lookup_body,
        out_type=jax.ShapeDtypeStruct((NS_, 8, 32, 8, 128), jnp.float32),
        mesh=mesh,
        scratch_types=[
            pltpu.VMEM((CPW * 128,), jnp.int32),
            pltpu.VMEM((3, 128, D), jnp.float32),
            pltpu.VMEM((2, 8, 8, 128), jnp.float32),
            pltpu.SemaphoreType.DMA((3,)),
            pltpu.SemaphoreType.DMA((2,)),
        ],
        compiler_params=pltpu.CompilerParams(use_tc_tiling_on_sc=False, needs_layout_passes=False),
    )
    return lookup(xt, tabl)


def kernel(x, embed_weight):
    xt = x.astype(jnp.int32).T.reshape(B)
    wt = embed_weight.T
    wtail = jnp.pad(embed_weight[VOCAB - 64:].T, ((0, 0), (0, 64)))
    out5 = _embed(xt, wt, wtail)
    return out5.transpose(2, 4, 0, 1, 3).reshape(NT, NS_, D)


# speedup vs baseline: 4.4622x; 1.0067x over previous
import math

import jax
import jax.numpy as jnp
from jax import lax
from jax.experimental import pallas as pl
from jax.experimental.pallas import tpu as pltpu
from jax.experimental.pallas import tpu_sc as plsc

VOCAB = 1000000
D = 64
NT = 4096
NS_ = 200
B = NT * NS_
SCALE = math.sqrt(D)

_info = plsc.get_sparse_core_info()
NC, NSUB, L = _info.num_cores, _info.num_subcores, _info.num_lanes
NW = NC * NSUB

FULL_BLOCKS = VOCAB // 128
BPW_BASE = FULL_BLOCKS // NW
BPW_EXTRA = FULL_BLOCKS - BPW_BASE * NW


def _repack_body(wt_hbm, wtail_hbm, tab_hbm, blk_v, tb_v, semg, sems):
    wid = lax.axis_index("s") * NC + lax.axis_index("c")
    iota = jax.lax.iota(jnp.int32, L)

    def fire_load(i, b):
        bl = wid + i * NW
        pltpu.async_copy(
            wt_hbm.at[:, pl.ds(pl.multiple_of(bl * 128, 128), 128)],
            blk_v.at[b], semg.at[b],
        )

    def transpose_blk(b, tb):
        for j0 in range(D // L):
            jlanes = iota + j0 * L

            @plsc.parallel_loop(0, 128, unroll=4)
            def transpose_col(r):
                cv = (iota + r) & 127
                v = plsc.load_gather(blk_v.at[b], [jlanes, cv])
                plsc.store_scatter(
                    tb_v.at[tb],
                    [jax.lax.shift_right_logical(cv, 1), (cv & 1) * D + jlanes],
                    v,
                )

    n_mine = jnp.where(wid < BPW_EXTRA, BPW_BASE + 1, BPW_BASE).astype(jnp.int32)

    fire_load(0, 0)

    @pl.when(1 < n_mine)
    def _():
        fire_load(1, 1)

    def block_step(i, _):
        b = i % 3
        bl = wid + i * NW

        @pl.when(i + 2 < n_mine)
        def _():
            fire_load(i + 2, (i + 2) % 3)

        pltpu.make_async_copy(
            wt_hbm.at[:, pl.ds(0, 128)], blk_v.at[b], semg.at[b]
        ).wait()

        tb = i & 1

        @pl.when(i >= 2)
        def _():
            pltpu.make_async_copy(
                tab_hbm.at[pl.ds(0, 64)], tb_v.at[tb], sems.at[tb]
            ).wait()

        transpose_blk(b, tb)
        pltpu.async_copy(
            tb_v.at[tb], tab_hbm.at[pl.ds(pl.multiple_of(bl * 64, 64), 64)],
            sems.at[tb],
        )
        return ()

    lax.fori_loop(0, n_mine, block_step, ())

    for b in range(2):
        pltpu.make_async_copy(
            tab_hbm.at[pl.ds(0, 64)], tb_v.at[b], sems.at[b]
        ).wait()

    @pl.when(wid == NW - 1)
    def _():
        pltpu.sync_copy(wtail_hbm, blk_v.at[0])
        transpose_blk(0, 0)
        pltpu.sync_copy(tb_v.at[0, pl.ds(0, 32)], tab_hbm.at[pl.ds(VOCAB // 2 - 32, 32)])


N_CHUNKS = B // 128
CPW = N_CHUNKS // NW


def _lookup_body(xt_hbm, tab_hbm, out_hbm, idx_v, g_v, tb_v, semg, sems):
    wid = lax.axis_index("s") * NC + lax.axis_index("c")
    iota = jax.lax.iota(jnp.int32, L)
    c0 = wid * CPW

    pltpu.sync_copy(xt_hbm.at[pl.ds(pl.multiple_of(c0 * 128, 128), CPW * 128)], idx_v)

    def fire_gather(i, g):
        pltpu.async_copy(
            tab_hbm.at[idx_v.at[pl.ds(pl.multiple_of(i * 128, 128), 128)]],
            g_v.at[g], semg.at[g],
        )

    fire_gather(0, 0)
    fire_gather(1, 1)
    fire_gather(2, 2)

    def step(i, _):
        g = i % 4
        b = i & 1

        @pl.when(i + 3 < CPW)
        def _():
            fire_gather(i + 3, (i + 3) % 4)

        pltpu.make_async_copy(
            tab_hbm.at[pl.ds(0, 128)], g_v.at[g], semg.at[g]
        ).wait()

        @pl.when(i >= 2)
        def _():
            pltpu.make_async_copy(
                out_hbm.at[0, :, 0], tb_v.at[b], sems.at[b]
            ).wait()

        for bl0 in range(8):
            rows = iota + bl0 * L

            @plsc.parallel_loop(0, D, unroll=4)
            def emit_j(j):
                jv = (iota + j) & (D - 1)
                v = plsc.load_gather(g_v.at[g], [rows, jv]) * SCALE
                plsc.store_scatter(
                    tb_v.at[b],
                    [jax.lax.shift_right_logical(jv, 3), jv & 7, rows],
                    v,
                )
        c = c0 + i
        pltpu.async_copy(tb_v.at[b], out_hbm.at[c // 32, :, c % 32], sems.at[b])
        return ()

    lax.fori_loop(0, CPW, step, ())

    for b in range(2):
        pltpu.make_async_copy(
            out_hbm.at[0, :, 0], tb_v.at[b], sems.at[b]
        ).wait()


@jax.jit
def _embed(xt, wt, wtail):
    mesh = plsc.VectorSubcoreMesh(core_axis_name="c", subcore_axis_name="s")
    repack = pl.kernel(
        _repack_body,
        out_type=jax.ShapeDtypeStruct((VOCAB // 2, 128), jnp.float32),
        mesh=mesh,
        scratch_types=[
            pltpu.VMEM((3, D, 128), jnp.float32),
            pltpu.VMEM((2, D, 128), jnp.float32),
            pltpu.SemaphoreType.DMA((3,)),
            pltpu.SemaphoreType.DMA((2,)),
        ],
        compiler_params=pltpu.CompilerParams(use_tc_tiling_on_sc=True, needs_layout_passes=False),
    )
    tab = repack(wt, wtail)
    tabl = tab.reshape(VOCAB, D)
    lookup = pl.kernel(
        _lookup_body,
        out_type=jax.ShapeDtypeStruct((NS_, 8, 32, 8, 128), jnp.float32),
        mesh=mesh,
        scratch_types=[
            pltpu.VMEM((CPW * 128,), jnp.int32),
            pltpu.VMEM((4, 128, D), jnp.float32),
            pltpu.VMEM((2, 8, 8, 128), jnp.float32),
            pltpu.SemaphoreType.DMA((4,)),
            pltpu.SemaphoreType.DMA((2,)),
        ],
        compiler_params=pltpu.CompilerParams(use_tc_tiling_on_sc=False, needs_layout_passes=False),
    )
    return lookup(xt, tabl)


def kernel(x, embed_weight):
    xt = x.astype(jnp.int32).T.reshape(B)
    wt = embed_weight.T
    wtail = jnp.pad(embed_weight[VOCAB - 64:].T, ((0, 0), (0, 64)))
    out5 = _embed(xt, wt, wtail)
    return out5.transpose(2, 4, 0, 1, 3).reshape(NT, NS_, D)


# speedup vs baseline: 4.6057x; 1.0322x over previous
import math

import jax
import jax.numpy as jnp
from jax import lax
from jax.experimental import pallas as pl
from jax.experimental.pallas import tpu as pltpu
from jax.experimental.pallas import tpu_sc as plsc

VOCAB = 1000000
D = 64
NT = 4096
NS_ = 200
B = NT * NS_
SCALE = math.sqrt(D)

_info = plsc.get_sparse_core_info()
NC, NSUB, L = _info.num_cores, _info.num_subcores, _info.num_lanes
NW = NC * NSUB

FULL_BLOCKS = VOCAB // 128
BPW_BASE = FULL_BLOCKS // NW
BPW_EXTRA = FULL_BLOCKS - BPW_BASE * NW


def _repack_body(wt_hbm, wtail_hbm, tab_hbm, blk_v, tb_v, semg, sems):
    wid = lax.axis_index("s") * NC + lax.axis_index("c")
    iota = jax.lax.iota(jnp.int32, L)

    def fire_load(i, b):
        bl = wid + i * NW
        pltpu.async_copy(
            wt_hbm.at[:, pl.ds(pl.multiple_of(bl * 128, 128), 128)],
            blk_v.at[b], semg.at[b],
        )

    def transpose_blk(b, tb):
        for j0 in range(D // L):
            jlanes = iota + j0 * L

            @plsc.parallel_loop(0, 128, unroll=4)
            def transpose_col(r):
                cv = (iota + r) & 127
                v = plsc.load_gather(blk_v.at[b], [jlanes, cv])
                plsc.store_scatter(
                    tb_v.at[tb],
                    [jax.lax.shift_right_logical(cv, 1), (cv & 1) * D + jlanes],
                    v,
                )

    n_mine = jnp.where(wid < BPW_EXTRA, BPW_BASE + 1, BPW_BASE).astype(jnp.int32)

    fire_load(0, 0)

    @pl.when(1 < n_mine)
    def _():
        fire_load(1, 1)

    @pl.when(2 < n_mine)
    def _():
        fire_load(2, 2)

    def block_step(i, _):
        b = i % 4
        bl = wid + i * NW

        @pl.when(i + 3 < n_mine)
        def _():
            fire_load(i + 3, (i + 3) % 4)

        pltpu.make_async_copy(
            wt_hbm.at[:, pl.ds(0, 128)], blk_v.at[b], semg.at[b]
        ).wait()

        tb = i & 1

        @pl.when(i >= 2)
        def _():
            pltpu.make_async_copy(
                tab_hbm.at[pl.ds(0, 64)], tb_v.at[tb], sems.at[tb]
            ).wait()

        transpose_blk(b, tb)
        pltpu.async_copy(
            tb_v.at[tb], tab_hbm.at[pl.ds(pl.multiple_of(bl * 64, 64), 64)],
            sems.at[tb],
        )
        return ()

    lax.fori_loop(0, n_mine, block_step, ())

    for b in range(2):
        pltpu.make_async_copy(
            tab_hbm.at[pl.ds(0, 64)], tb_v.at[b], sems.at[b]
        ).wait()

    @pl.when(wid == NW - 1)
    def _():
        pltpu.sync_copy(wtail_hbm, blk_v.at[0])
        transpose_blk(0, 0)
        pltpu.sync_copy(tb_v.at[0, pl.ds(0, 32)], tab_hbm.at[pl.ds(VOCAB // 2 - 32, 32)])


N_CHUNKS = B // 128
CPW = N_CHUNKS // NW


def _lookup_body(xt_hbm, tab_hbm, out_hbm, idx_v, g_v, tb_v, semg, sems):
    wid = lax.axis_index("s") * NC + lax.axis_index("c")
    iota = jax.lax.iota(jnp.int32, L)
    c0 = wid * CPW

    pltpu.sync_copy(xt_hbm.at[pl.ds(pl.multiple_of(c0 * 128, 128), CPW * 128)], idx_v)

    def fire_gather(i, g):
        pltpu.async_copy(
            tab_hbm.at[idx_v.at[pl.ds(pl.multiple_of(i * 128, 128), 128)]],
            g_v.at[g], semg.at[g],
        )

    fire_gather(0, 0)
    fire_gather(1, 1)
    fire_gather(2, 2)

    def step(i, _):
        g = i % 4
        b = i & 1

        @pl.when(i + 3 < CPW)
        def _():
            fire_gather(i + 3, (i + 3) % 4)

        pltpu.make_async_copy(
            tab_hbm.at[pl.ds(0, 128)], g_v.at[g], semg.at[g]
        ).wait()

        @pl.when(i >= 2)
        def _():
            pltpu.make_async_copy(
                out_hbm.at[0, :, 0], tb_v.at[b], sems.at[b]
            ).wait()

        for bl0 in range(8):
            rows = iota + bl0 * L

            @plsc.parallel_loop(0, D, unroll=8)
            def emit_j(j):
                jv = (iota + j) & (D - 1)
                v = plsc.load_gather(g_v.at[g], [rows, jv]) * SCALE
                plsc.store_scatter(
                    tb_v.at[b],
                    [jax.lax.shift_right_logical(jv, 3), jv & 7, rows],
                    v,
                )
        c = c0 + i
        pltpu.async_copy(tb_v.at[b], out_hbm.at[c // 32, :, c % 32], sems.at[b])
        return ()

    lax.fori_loop(0, CPW, step, ())

    for b in range(2):
        pltpu.make_async_copy(
            out_hbm.at[0, :, 0], tb_v.at[b], sems.at[b]
        ).wait()


@jax.jit
def _embed(xt, wt, wtail):
    mesh = plsc.VectorSubcoreMesh(core_axis_name="c", subcore_axis_name="s")
    repack = pl.kernel(
        _repack_body,
        out_type=jax.ShapeDtypeStruct((VOCAB // 2, 128), jnp.float32),
        mesh=mesh,
        scratch_types=[
            pltpu.VMEM((4, D, 128), jnp.float32),
            pltpu.VMEM((2, D, 128), jnp.float32),
            pltpu.SemaphoreType.DMA((4,)),
            pltpu.SemaphoreType.DMA((2,)),
        ],
        compiler_params=pltpu.CompilerParams(use_tc_tiling_on_sc=True, needs_layout_passes=False),
    )
    tab = repack(wt, wtail)
    tabl = tab.reshape(VOCAB, D)
    lookup = pl.kernel(
        _lookup_body,
        out_type=jax.ShapeDtypeStruct((NS_, 8, 32, 8, 128), jnp.float32),
        mesh=mesh,
        scratch_types=[
            pltpu.VMEM((CPW * 128,), jnp.int32),
            pltpu.VMEM((4, 128, D), jnp.float32),
            pltpu.VMEM((2, 8, 8, 128), jnp.float32),
            pltpu.SemaphoreType.DMA((4,)),
            pltpu.SemaphoreType.DMA((2,)),
        ],
        compiler_params=pltpu.CompilerParams(use_tc_tiling_on_sc=False, needs_layout_passes=False),
    )
    return lookup(xt, tabl)


def kernel(x, embed_weight):
    xt = x.astype(jnp.int32).T.reshape(B)
    wt = embed_weight.T
    wtail = jnp.pad(embed_weight[VOCAB - 64:].T, ((0, 0), (0, 64)))
    out5 = _embed(xt, wt, wtail)
    return out5.transpose(2, 4, 0, 1, 3).reshape(NT, NS_, D)


# speedup vs baseline: 4.6250x; 1.0042x over previous
import math

import jax
import jax.numpy as jnp
from jax import lax
from jax.experimental import pallas as pl
from jax.experimental.pallas import tpu as pltpu
from jax.experimental.pallas import tpu_sc as plsc

VOCAB = 1000000
D = 64
NT = 4096
NS_ = 200
B = NT * NS_
SCALE = math.sqrt(D)

_info = plsc.get_sparse_core_info()
NC, NSUB, L = _info.num_cores, _info.num_subcores, _info.num_lanes
NW = NC * NSUB

FULL_BLOCKS = VOCAB // 128
BPW_BASE = FULL_BLOCKS // NW
BPW_EXTRA = FULL_BLOCKS - BPW_BASE * NW


def _repack_body(wt_hbm, wtail_hbm, tab_hbm, blk_v, tb_v, semg, sems):
    wid = lax.axis_index("s") * NC + lax.axis_index("c")
    iota = jax.lax.iota(jnp.int32, L)

    def fire_load(i, b):
        bl = wid + i * NW
        pltpu.async_copy(
            wt_hbm.at[:, pl.ds(pl.multiple_of(bl * 128, 128), 128)],
            blk_v.at[b], semg.at[b],
        )

    def transpose_blk(b, tb):
        for j0 in range(D // L):
            jlanes = iota + j0 * L

            @plsc.parallel_loop(0, 128, unroll=8)
            def transpose_col(r):
                cv = (iota + r) & 127
                v = plsc.load_gather(blk_v.at[b], [jlanes, cv])
                plsc.store_scatter(
                    tb_v.at[tb],
                    [jax.lax.shift_right_logical(cv, 1), (cv & 1) * D + jlanes],
                    v,
                )

    n_mine = jnp.where(wid < BPW_EXTRA, BPW_BASE + 1, BPW_BASE).astype(jnp.int32)

    fire_load(0, 0)

    @pl.when(1 < n_mine)
    def _():
        fire_load(1, 1)

    @pl.when(2 < n_mine)
    def _():
        fire_load(2, 2)

    def block_step(i, _):
        b = i % 4
        bl = wid + i * NW

        @pl.when(i + 3 < n_mine)
        def _():
            fire_load(i + 3, (i + 3) % 4)

        pltpu.make_async_copy(
            wt_hbm.at[:, pl.ds(0, 128)], blk_v.at[b], semg.at[b]
        ).wait()

        tb = i & 1

        @pl.when(i >= 2)
        def _():
            pltpu.make_async_copy(
                tab_hbm.at[pl.ds(0, 64)], tb_v.at[tb], sems.at[tb]
            ).wait()

        transpose_blk(b, tb)
        pltpu.async_copy(
            tb_v.at[tb], tab_hbm.at[pl.ds(pl.multiple_of(bl * 64, 64), 64)],
            sems.at[tb],
        )
        return ()

    lax.fori_loop(0, n_mine, block_step, ())

    for b in range(2):
        pltpu.make_async_copy(
            tab_hbm.at[pl.ds(0, 64)], tb_v.at[b], sems.at[b]
        ).wait()

    @pl.when(wid == NW - 1)
    def _():
        pltpu.sync_copy(wtail_hbm, blk_v.at[0])
        transpose_blk(0, 0)
        pltpu.sync_copy(tb_v.at[0, pl.ds(0, 32)], tab_hbm.at[pl.ds(VOCAB // 2 - 32, 32)])


N_CHUNKS = B // 128
CPW = N_CHUNKS // NW


def _lookup_body(xt_hbm, tab_hbm, out_hbm, idx_v, g_v, tb_v, semg, sems):
    wid = lax.axis_index("s") * NC + lax.axis_index("c")
    iota = jax.lax.iota(jnp.int32, L)
    c0 = wid * CPW

    pltpu.sync_copy(xt_hbm.at[pl.ds(pl.multiple_of(c0 * 128, 128), CPW * 128)], idx_v)

    def fire_gather(i, g):
        pltpu.async_copy(
            tab_hbm.at[idx_v.at[pl.ds(pl.multiple_of(i * 128, 128), 128)]],
            g_v.at[g], semg.at[g],
        )

    fire_gather(0, 0)
    fire_gather(1, 1)
    fire_gather(2, 2)

    def step(i, _):
        g = i % 4
        b = i & 1

        @pl.when(i + 3 < CPW)
        def _():
            fire_gather(i + 3, (i + 3) % 4)

        pltpu.make_async_copy(
            tab_hbm.at[pl.ds(0, 128)], g_v.at[g], semg.at[g]
        ).wait()

        @pl.when(i >= 2)
        def _():
            pltpu.make_async_copy(
                out_hbm.at[0, :, 0], tb_v.at[b], sems.at[b]
            ).wait()

        for bl0 in range(8):
            rows = iota + bl0 * L

            @plsc.parallel_loop(0, D, unroll=8)
            def emit_j(j):
                jv = (iota + j) & (D - 1)
                v = plsc.load_gather(g_v.at[g], [rows, jv]) * SCALE
                plsc.store_scatter(
                    tb_v.at[b],
                    [jax.lax.shift_right_logical(jv, 3), jv & 7, rows],
                    v,
                )
        c = c0 + i
        pltpu.async_copy(tb_v.at[b], out_hbm.at[c // 32, :, c % 32], sems.at[b])
        return ()

    lax.fori_loop(0, CPW, step, ())

    for b in range(2):
        pltpu.make_async_copy(
            out_hbm.at[0, :, 0], tb_v.at[b], sems.at[b]
        ).wait()


@jax.jit
def _embed(xt, wt, wtail):
    mesh = plsc.VectorSubcoreMesh(core_axis_name="c", subcore_axis_name="s")
    repack = pl.kernel(
        _repack_body,
        out_type=jax.ShapeDtypeStruct((VOCAB // 2, 128), jnp.float32),
        mesh=mesh,
        scratch_types=[
            pltpu.VMEM((4, D, 128), jnp.float32),
            pltpu.VMEM((2, D, 128), jnp.float32),
            pltpu.SemaphoreType.DMA((4,)),
            pltpu.SemaphoreType.DMA((2,)),
        ],
        compiler_params=pltpu.CompilerParams(use_tc_tiling_on_sc=True, needs_layout_passes=False),
    )
    tab = repack(wt, wtail)
    tabl = tab.reshape(VOCAB, D)
    lookup = pl.kernel(
        _lookup_body,
        out_type=jax.ShapeDtypeStruct((NS_, 8, 32, 8, 128), jnp.float32),
        mesh=mesh,
        scratch_types=[
            pltpu.VMEM((CPW * 128,), jnp.int32),
            pltpu.VMEM((4, 128, D), jnp.float32),
            pltpu.VMEM((2, 8, 8, 128), jnp.float32),
            pltpu.SemaphoreType.DMA((4,)),
            pltpu.SemaphoreType.DMA((2,)),
        ],
        compiler_params=pltpu.CompilerParams(use_tc_tiling_on_sc=False, needs_layout_passes=False),
    )
    return lookup(xt, tabl)


def kernel(x, embed_weight):
    xt = x.astype(jnp.int32).T.reshape(B)
    wt = embed_weight.T
    wtail = jnp.pad(embed_weight[VOCAB - 64:].T, ((0, 0), (0, 64)))
    out5 = _embed(xt, wt, wtail)
    return out5.transpose(2, 4, 0, 1, 3).reshape(NT, NS_, D)
